# Initial kernel scaffold; baseline (speedup 1.0000x reference)
import jax
import jax.numpy as jnp
from jax.experimental import pallas as pl


def kernel(edge_index, edge_weight, users_w, items_w):
    raise NotImplementedError("write your pallas kernel here")


# speedup vs baseline: 6.7270x; 6.7270x over previous
import functools

import jax
import jax.numpy as jnp
from jax import lax
from jax.experimental import pallas as pl
from jax.experimental.pallas import tpu as pltpu
from jax.experimental.pallas import tpu_sc as plsc

NUM_USERS = 25000
NUM_ITEMS = 25000
N = NUM_USERS + NUM_ITEMS
EMB = 64
HALF = EMB // 2
K = 3
E = 800000
CH = 128
NT = 16
NCH_TILE = 392
EP = NCH_TILE * CH * NT
NCHUNKS = EP // CH
GROUPS = NCH_TILE // 4
NP = 50048
STRIPE = NP // NT
ZROWS = 184
ZBLKS = STRIPE // ZROWS


def _sc_body(packed, wtab, t0, e1, e2, e3,
             ib0, ib1, ib2, ib3, wb0, wb1, wb2, wb3, rb0, rb1, zbuf, acc,
             is0, is1, is2, is3, gs0, gs1, ss0, ss1):
    c = lax.axis_index("c")
    s = lax.axis_index("s")
    ibufs = (ib0, ib1, ib2, ib3)
    wbufs = (wb0, wb1, wb2, wb3)
    rbufs = (rb0, rb1)
    isems = (is0, is1, is2, is3)
    gsems = (gs0, gs1)
    ssems = (ss0, ss1)

    z16 = jnp.zeros((16,), jnp.float32)
    coff = jnp.broadcast_to(c * NP, (16,)).astype(jnp.int32)
    base = s * NCH_TILE
    stripe0 = s * STRIPE
    cN = c * NP

    def _zb(r, carry):
        zbuf[r, pl.ds(0, 16)] = z16
        zbuf[r, pl.ds(16, 16)] = z16
        return carry
    lax.fori_loop(0, ZROWS, _zb, 0, unroll=4)
    for z in range(ZBLKS):
        pltpu.sync_copy(zbuf, acc.at[pl.ds(stripe0 + z * ZROWS, ZROWS)])
    plsc.subcore_barrier()

    def fix_col(ib):
        for v in range(CH // 16):
            sl = pl.ds(v * 16, 16)
            ib[0, sl] = ib[0, sl] + coff

    def scale(rb, wb):
        def body(r, carry):
            wv = plsc.load_gather(wb, [jnp.broadcast_to(r, (16,))])
            a0 = rb[r, pl.ds(0, 16)]
            rb[r, pl.ds(0, 16)] = a0 * wv
            a1 = rb[r, pl.ds(16, 16)]
            rb[r, pl.ds(16, 16)] = a1 * wv
            return carry
        lax.fori_loop(0, CH, body, 0, unroll=4)

    for k, (tsrc, tdst) in enumerate(((t0, e1), (e1, e2), (e2, e3))):
        for b in range(3):
            pltpu.async_copy(packed.at[base + b], ibufs[b], isems[b])
            pltpu.async_copy(wtab.at[base + b], wbufs[b], isems[b])
        pltpu.make_async_copy(packed.at[base], ibufs[0], isems[0]).wait()
        pltpu.make_async_copy(wtab.at[base], wbufs[0], isems[0]).wait()
        fix_col(ibufs[0])
        pltpu.async_copy(tsrc.at[ibufs[0].at[0]], rbufs[0], gsems[0])

        def group(g, carry):
            for b in range(4):
                i = 4 * g + b
                bn1, bn3 = (b + 1) % 4, (b + 3) % 4
                rn1 = (b + 1) % 2
                pltpu.make_async_copy(
                    packed.at[base], ibufs[bn1], isems[bn1]).wait()
                pltpu.make_async_copy(
                    wtab.at[base], wbufs[bn1], isems[bn1]).wait()
                fix_col(ibufs[bn1])
                if b == 0:
                    @pl.when(g > 0)
                    def _():
                        pltpu.make_async_copy(
                            rbufs[rn1], acc.at[ibufs[bn3].at[1]],
                            ssems[rn1]).wait()
                else:
                    pltpu.make_async_copy(
                        rbufs[rn1], acc.at[ibufs[bn3].at[1]],
                        ssems[rn1]).wait()
                cid = base + jnp.minimum(i + 3, NCH_TILE - 1)
                pltpu.async_copy(packed.at[cid], ibufs[bn3], isems[bn3])
                pltpu.async_copy(wtab.at[cid], wbufs[bn3], isems[bn3])
                pltpu.async_copy(
                    tsrc.at[ibufs[bn1].at[0]], rbufs[rn1], gsems[rn1])
                pltpu.make_async_copy(
                    tsrc.at[ibufs[b].at[0]], rbufs[b % 2],
                    gsems[b % 2]).wait()
                scale(rbufs[b % 2], wbufs[b])
                pltpu.async_copy(
                    rbufs[b % 2], acc.at[ibufs[b].at[1]], ssems[b % 2],
                    add=True)
            return carry
        lax.fori_loop(0, GROUPS, group, 0)

        pltpu.make_async_copy(rbufs[1], acc.at[ibufs[3].at[1]], ssems[1]).wait()
        pltpu.make_async_copy(tsrc.at[ibufs[0].at[0]], rbufs[0],
                              gsems[0]).wait()
        pltpu.make_async_copy(packed.at[base], ibufs[1], isems[1]).wait()
        pltpu.make_async_copy(wtab.at[base], wbufs[1], isems[1]).wait()
        pltpu.make_async_copy(packed.at[base], ibufs[2], isems[2]).wait()
        pltpu.make_async_copy(wtab.at[base], wbufs[2], isems[2]).wait()
        plsc.subcore_barrier()

        for z in range(ZBLKS):
            sl = pl.ds(stripe0 + z * ZROWS, ZROWS)
            pltpu.sync_copy(acc.at[sl],
                            tdst.at[pl.ds(cN + stripe0 + z * ZROWS, ZROWS)])
            if k < K - 1:
                pltpu.sync_copy(zbuf, acc.at[sl])
        plsc.subcore_barrier()


_sc_prop = functools.partial(
    pl.kernel,
    out_type=[jax.ShapeDtypeStruct((2 * NP, HALF), jnp.float32)] * 3,
    mesh=plsc.VectorSubcoreMesh(core_axis_name="c", subcore_axis_name="s"),
    compiler_params=pltpu.CompilerParams(
        needs_layout_passes=False, use_tc_tiling_on_sc=False),
    scratch_types=[pltpu.VMEM((2, CH), jnp.int32)] * 4
                  + [pltpu.VMEM((CH,), jnp.float32)] * 4
                  + [pltpu.VMEM((CH, HALF), jnp.float32)] * 2
                  + [pltpu.VMEM((ZROWS, HALF), jnp.float32)]
                  + [pltpu.VMEM_SHARED((NP, HALF), jnp.float32)]
                  + [pltpu.SemaphoreType.DMA] * 8,
)(_sc_body)


def _tc_mean_body(al, bl, cl, dl, ah, bh, ch_, dh, o):
    o[:, :HALF] = (al[...] + bl[...] + cl[...] + dl[...]) * 0.25
    o[:, HALF:] = (ah[...] + bh[...] + ch_[...] + dh[...]) * 0.25


_MROWS = 400
_MBLKS = N // _MROWS


def _tc_mean(los, his):
    spec = pl.BlockSpec((_MROWS, HALF), lambda i: (i, 0))
    return pl.pallas_call(
        _tc_mean_body,
        grid=(_MBLKS,),
        in_specs=[spec] * 8,
        out_specs=pl.BlockSpec((_MROWS, EMB), lambda i: (i, 0)),
        out_shape=jax.ShapeDtypeStruct((N, EMB), jnp.float32),
    )(*los, *his)


def kernel(edge_index, edge_weight, users_w, items_w):
    col = edge_index[1].astype(jnp.int32)
    row = edge_index[0].astype(jnp.int32)
    w = edge_weight.astype(jnp.float32)
    pad = EP - E
    col = jnp.concatenate([col, jnp.zeros((pad,), jnp.int32)])
    row = jnp.concatenate([row, jnp.zeros((pad,), jnp.int32)])
    w = jnp.concatenate([w, jnp.zeros((pad,), jnp.float32)])
    packed = jnp.stack(
        [col.reshape(NCHUNKS, CH), row.reshape(NCHUNKS, CH)],
        axis=1)
    wtab = w.reshape(NCHUNKS, CH)
    emb0 = jnp.concatenate(
        [users_w, items_w, jnp.zeros((NP - N, EMB), jnp.float32)], axis=0)
    t0 = jnp.concatenate([emb0[:, :HALF], emb0[:, HALF:]], axis=0)
    e1, e2, e3 = _sc_prop(packed, wtab, t0)
    los = (t0, e1, e2, e3)
    his = tuple(a[NP:NP + N] for a in los)
    final = _tc_mean(los, his)
    return final[:NUM_USERS], users_w, final[NUM_USERS:], items_w


# speedup vs baseline: 7.2193x; 1.0732x over previous
import functools

import jax
import jax.numpy as jnp
from jax import lax
from jax.experimental import pallas as pl
from jax.experimental.pallas import tpu as pltpu
from jax.experimental.pallas import tpu_sc as plsc

NUM_USERS = 25000
NUM_ITEMS = 25000
N = NUM_USERS + NUM_ITEMS
EMB = 64
HALF = EMB // 2
K = 3
E = 800000
CH = 128
NT = 16
NCH_TILE = 392
EP = NCH_TILE * CH * NT
NCHUNKS = EP // CH
GROUPS = NCH_TILE // 4
NP = 50176
STRIPE = NP // NT
ZROWS = 112
ZBLKS = STRIPE // ZROWS
ZV = ZROWS // 16


def _sc_body(packed, wtab, t0, favg, e1, e2,
             ib0, ib1, ib2, ib3, wb0, wb1, wb2, wb3, rb0, rb1, zbuf, acc,
             ta0, tb0, ta1, tb1, idxb,
             is0, is1, is2, is3, gs0, gs1, ss0, ss1, ms0, ms1):
    c = lax.axis_index("c")
    s = lax.axis_index("s")
    ibufs = (ib0, ib1, ib2, ib3)
    wbufs = (wb0, wb1, wb2, wb3)
    rbufs = (rb0, rb1)
    isems = (is0, is1, is2, is3)
    gsems = (gs0, gs1)
    ssems = (ss0, ss1)
    msems = (ms0, ms1)

    z16 = jnp.zeros((16,), jnp.float32)
    qtr = jnp.full((16,), 0.25, jnp.float32)
    iota16 = lax.iota(jnp.int32, 16)
    coff = jnp.broadcast_to(c * NP, (16,)).astype(jnp.int32)
    base = s * NCH_TILE
    stripe0 = s * STRIPE
    cN = c * NP

    def _zb(r, carry):
        zbuf[r, pl.ds(0, 16)] = z16
        zbuf[r, pl.ds(16, 16)] = z16
        return carry
    lax.fori_loop(0, ZROWS, _zb, 0, unroll=4)
    for z in range(ZBLKS):
        pltpu.sync_copy(zbuf, acc.at[pl.ds(stripe0 + z * ZROWS, ZROWS)])
    plsc.subcore_barrier()

    def prep(ib, wb, last):
        for v in range(CH // 16):
            sl = pl.ds(v * 16, 16)
            ib[0, sl] = ib[0, sl] + coff
            if last:
                wb[sl] = wb[sl] * qtr

    def scale(rb, wb):
        def body(r, carry):
            wv = plsc.load_gather(wb, [jnp.broadcast_to(r, (16,))])
            a0 = rb[r, pl.ds(0, 16)]
            rb[r, pl.ds(0, 16)] = a0 * wv
            a1 = rb[r, pl.ds(16, 16)]
            rb[r, pl.ds(16, 16)] = a1 * wv
            return carry
        lax.fori_loop(0, CH, body, 0, unroll=4)

    def mean_add_phase():
        tas = (ta0, ta1)
        tbs = (tb0, tb1)

        def addvec(dst, src):
            def body(r, carry):
                dst[r, pl.ds(0, 16)] = (dst[r, pl.ds(0, 16)]
                                        + src[r, pl.ds(0, 16)])
                dst[r, pl.ds(16, 16)] = (dst[r, pl.ds(16, 16)]
                                         + src[r, pl.ds(16, 16)])
                return carry
            lax.fori_loop(0, ZROWS, body, 0, unroll=4)

        def addvec_q(dst, src):
            def body(r, carry):
                dst[r, pl.ds(0, 16)] = (dst[r, pl.ds(0, 16)]
                                        + src[r, pl.ds(0, 16)]) * qtr
                dst[r, pl.ds(16, 16)] = (dst[r, pl.ds(16, 16)]
                                         + src[r, pl.ds(16, 16)]) * qtr
                return carry
            lax.fori_loop(0, ZROWS, body, 0, unroll=4)

        def issue_inputs(z, p):
            slh = pl.ds(cN + stripe0 + z * ZROWS, ZROWS)
            pltpu.async_copy(t0.at[slh], tas[p], msems[p])
            pltpu.async_copy(e1.at[slh], tbs[p], msems[p])

        def handle(z, p):
            slh = pl.ds(cN + stripe0 + z * ZROWS, ZROWS)
            pltpu.make_async_copy(t0.at[slh], tas[p], msems[p]).wait()
            pltpu.make_async_copy(e1.at[slh], tbs[p], msems[p]).wait()
            addvec(tas[p], tbs[p])
            pltpu.async_copy(e2.at[slh], tbs[p], msems[p])
            pltpu.make_async_copy(e2.at[slh], tbs[p], msems[p]).wait()
            addvec_q(tas[p], tbs[p])
            rbase = stripe0 + z * ZROWS
            for v in range(ZV):
                idxb[pl.ds(v * 16, 16)] = iota16 + (rbase + v * 16)
            pltpu.sync_copy(tas[p], acc.at[idxb], add=True)
            znxt = jnp.minimum(z + 2, ZBLKS - 1)
            issue_inputs(znxt, p)

        issue_inputs(0, 0)
        issue_inputs(1, 1)

        def mgrp(g, carry):
            handle(2 * g, 0)
            handle(2 * g + 1, 1)
            return carry
        lax.fori_loop(0, ZBLKS // 2, mgrp, 0)
        for p in range(2):
            slh = pl.ds(cN + stripe0 + (ZBLKS - 1) * ZROWS, ZROWS)
            pltpu.make_async_copy(t0.at[slh], tas[p], msems[p]).wait()
            pltpu.make_async_copy(e1.at[slh], tbs[p], msems[p]).wait()

    for k, (tsrc, tdst) in enumerate(((t0, e1), (e1, e2), (e2, None))):
        last = k == K - 1
        if last:
            mean_add_phase()
        for b in range(3):
            pltpu.async_copy(packed.at[base + b], ibufs[b], isems[b])
            pltpu.async_copy(wtab.at[base + b], wbufs[b], isems[b])
        pltpu.make_async_copy(packed.at[base], ibufs[0], isems[0]).wait()
        pltpu.make_async_copy(wtab.at[base], wbufs[0], isems[0]).wait()
        prep(ibufs[0], wbufs[0], last)
        pltpu.async_copy(tsrc.at[ibufs[0].at[0]], rbufs[0], gsems[0])

        def group(g, carry):
            for b in range(4):
                i = 4 * g + b
                bn1, bn3 = (b + 1) % 4, (b + 3) % 4
                rn1 = (b + 1) % 2
                pltpu.make_async_copy(
                    packed.at[base], ibufs[bn1], isems[bn1]).wait()
                pltpu.make_async_copy(
                    wtab.at[base], wbufs[bn1], isems[bn1]).wait()
                prep(ibufs[bn1], wbufs[bn1], last)
                if b == 0:
                    @pl.when(g > 0)
                    def _():
                        pltpu.make_async_copy(
                            rbufs[rn1], acc.at[ibufs[bn3].at[1]],
                            ssems[rn1]).wait()
                else:
                    pltpu.make_async_copy(
                        rbufs[rn1], acc.at[ibufs[bn3].at[1]],
                        ssems[rn1]).wait()
                cid = base + jnp.minimum(i + 3, NCH_TILE - 1)
                pltpu.async_copy(packed.at[cid], ibufs[bn3], isems[bn3])
                pltpu.async_copy(wtab.at[cid], wbufs[bn3], isems[bn3])
                pltpu.async_copy(
                    tsrc.at[ibufs[bn1].at[0]], rbufs[rn1], gsems[rn1])
                pltpu.make_async_copy(
                    tsrc.at[ibufs[b].at[0]], rbufs[b % 2],
                    gsems[b % 2]).wait()
                scale(rbufs[b % 2], wbufs[b])
                pltpu.async_copy(
                    rbufs[b % 2], acc.at[ibufs[b].at[1]], ssems[b % 2],
                    add=True)
            return carry
        lax.fori_loop(0, GROUPS, group, 0)

        pltpu.make_async_copy(rbufs[1], acc.at[ibufs[3].at[1]], ssems[1]).wait()
        pltpu.make_async_copy(tsrc.at[ibufs[0].at[0]], rbufs[0],
                              gsems[0]).wait()
        pltpu.make_async_copy(packed.at[base], ibufs[1], isems[1]).wait()
        pltpu.make_async_copy(wtab.at[base], wbufs[1], isems[1]).wait()
        pltpu.make_async_copy(packed.at[base], ibufs[2], isems[2]).wait()
        pltpu.make_async_copy(wtab.at[base], wbufs[2], isems[2]).wait()
        plsc.subcore_barrier()

        out_ref = favg if last else tdst
        for z in range(ZBLKS):
            sl = pl.ds(stripe0 + z * ZROWS, ZROWS)
            pltpu.sync_copy(acc.at[sl],
                            out_ref.at[pl.ds(cN + stripe0 + z * ZROWS, ZROWS)])
            if not last:
                pltpu.sync_copy(zbuf, acc.at[sl])
        plsc.subcore_barrier()


_sc_prop = functools.partial(
    pl.kernel,
    out_type=[jax.ShapeDtypeStruct((2 * NP, HALF), jnp.float32)] * 3,
    mesh=plsc.VectorSubcoreMesh(core_axis_name="c", subcore_axis_name="s"),
    compiler_params=pltpu.CompilerParams(
        needs_layout_passes=False, use_tc_tiling_on_sc=False),
    scratch_types=[pltpu.VMEM((2, CH), jnp.int32)] * 4
                  + [pltpu.VMEM((CH,), jnp.float32)] * 4
                  + [pltpu.VMEM((CH, HALF), jnp.float32)] * 2
                  + [pltpu.VMEM((ZROWS, HALF), jnp.float32)]
                  + [pltpu.VMEM_SHARED((NP, HALF), jnp.float32)]
                  + [pltpu.VMEM((ZROWS, HALF), jnp.float32)] * 4
                  + [pltpu.VMEM((ZROWS,), jnp.int32)]
                  + [pltpu.SemaphoreType.DMA] * 10,
)(_sc_body)


def kernel(edge_index, edge_weight, users_w, items_w):
    col = edge_index[1].astype(jnp.int32)
    row = edge_index[0].astype(jnp.int32)
    w = edge_weight.astype(jnp.float32)
    pad = EP - E
    col = jnp.concatenate([col, jnp.zeros((pad,), jnp.int32)])
    row = jnp.concatenate([row, jnp.zeros((pad,), jnp.int32)])
    w = jnp.concatenate([w, jnp.zeros((pad,), jnp.float32)])
    packed = jnp.stack(
        [col.reshape(NCHUNKS, CH), row.reshape(NCHUNKS, CH)],
        axis=1)
    wtab = w.reshape(NCHUNKS, CH)
    emb0 = jnp.concatenate(
        [users_w, items_w, jnp.zeros((NP - N, EMB), jnp.float32)], axis=0)
    t0 = jnp.concatenate([emb0[:, :HALF], emb0[:, HALF:]], axis=0)
    favg, _e1, _e2 = _sc_prop(packed, wtab, t0)
    final = jnp.concatenate([favg[:N], favg[NP:NP + N]], axis=1)
    return final[:NUM_USERS], users_w, final[NUM_USERS:], items_w


# speedup vs baseline: 9.5402x; 1.3215x over previous
import functools

import jax
import jax.numpy as jnp
from jax import lax
from jax.experimental import pallas as pl
from jax.experimental.pallas import tpu as pltpu
from jax.experimental.pallas import tpu_sc as plsc

NUM_USERS = 25000
NUM_ITEMS = 25000
N = NUM_USERS + NUM_ITEMS
EMB = 64
HALF = EMB // 2
K = 3
E = 800000
CH = 128
NT = 16
NCH_TILE = 392
EP = NCH_TILE * CH * NT
NCHUNKS = EP // CH
GROUPS = NCH_TILE // 4
NP = 50176
STRIPE = NP // NT
ZROWS = 112
ZBLKS = STRIPE // ZROWS
ZV = ZROWS // 16


def _sc_body(packed, wtab, t0, favg, e1, e2,
             ib0, ib1, ib2, ib3, wb0, wb1, wb2, wb3, rb0, rb1, zbuf, acc,
             ta0, tb0, ta1, tb1, idxb,
             is0, is1, is2, is3, gs0, gs1, ss0, ss1, ms0, ms1):
    c = lax.axis_index("c")
    s = lax.axis_index("s")
    ibufs = (ib0, ib1, ib2, ib3)
    wbufs = (wb0, wb1, wb2, wb3)
    rbufs = (rb0, rb1)
    isems = (is0, is1, is2, is3)
    gsems = (gs0, gs1)
    ssems = (ss0, ss1)
    msems = (ms0, ms1)

    z16 = jnp.zeros((16,), jnp.float32)
    qtr = jnp.full((16,), 0.25, jnp.float32)
    iota16 = lax.iota(jnp.int32, 16)
    coff = jnp.broadcast_to(c * NP, (16,)).astype(jnp.int32)
    base = s * NCH_TILE
    stripe0 = s * STRIPE
    cN = c * NP

    def _zb(r, carry):
        zbuf[r, pl.ds(0, 16)] = z16
        zbuf[r, pl.ds(16, 16)] = z16
        return carry
    lax.fori_loop(0, ZROWS, _zb, 0, unroll=4)
    for z in range(ZBLKS):
        pltpu.sync_copy(zbuf, acc.at[pl.ds(stripe0 + z * ZROWS, ZROWS)])
    plsc.subcore_barrier()

    def prep(ib, wb, last):
        for v in range(CH // 16):
            sl = pl.ds(v * 16, 16)
            ib[0, sl] = ib[0, sl] + coff
            if last:
                wb[sl] = wb[sl] * qtr

    _GDN = lax.GatherDimensionNumbers(
        offset_dims=(), collapsed_slice_dims=(0,), start_index_map=(0,))

    def scale(rb, wb):
        def body(g, carry):
            w16 = wb[pl.ds(g * 16, 16)]
            for j in range(16):
                r = g * 16 + j
                wv = lax.gather(w16, jnp.full((16, 1), j, jnp.int32),
                                dimension_numbers=_GDN, slice_sizes=(1,),
                                mode=lax.GatherScatterMode.PROMISE_IN_BOUNDS)
                a0 = rb[r, pl.ds(0, 16)]
                rb[r, pl.ds(0, 16)] = a0 * wv
                a1 = rb[r, pl.ds(16, 16)]
                rb[r, pl.ds(16, 16)] = a1 * wv
            return carry
        lax.fori_loop(0, CH // 16, body, 0)

    def mean_add_phase():
        tas = (ta0, ta1)
        tbs = (tb0, tb1)

        def addvec(dst, src):
            def body(r, carry):
                dst[r, pl.ds(0, 16)] = (dst[r, pl.ds(0, 16)]
                                        + src[r, pl.ds(0, 16)])
                dst[r, pl.ds(16, 16)] = (dst[r, pl.ds(16, 16)]
                                         + src[r, pl.ds(16, 16)])
                return carry
            lax.fori_loop(0, ZROWS, body, 0, unroll=4)

        def addvec_q(dst, src):
            def body(r, carry):
                dst[r, pl.ds(0, 16)] = (dst[r, pl.ds(0, 16)]
                                        + src[r, pl.ds(0, 16)]) * qtr
                dst[r, pl.ds(16, 16)] = (dst[r, pl.ds(16, 16)]
                                         + src[r, pl.ds(16, 16)]) * qtr
                return carry
            lax.fori_loop(0, ZROWS, body, 0, unroll=4)

        def issue_inputs(z, p):
            slh = pl.ds(cN + stripe0 + z * ZROWS, ZROWS)
            pltpu.async_copy(t0.at[slh], tas[p], msems[p])
            pltpu.async_copy(e1.at[slh], tbs[p], msems[p])

        def handle(z, p):
            slh = pl.ds(cN + stripe0 + z * ZROWS, ZROWS)
            pltpu.make_async_copy(t0.at[slh], tas[p], msems[p]).wait()
            pltpu.make_async_copy(e1.at[slh], tbs[p], msems[p]).wait()
            addvec(tas[p], tbs[p])
            pltpu.async_copy(e2.at[slh], tbs[p], msems[p])
            pltpu.make_async_copy(e2.at[slh], tbs[p], msems[p]).wait()
            addvec_q(tas[p], tbs[p])
            rbase = stripe0 + z * ZROWS
            for v in range(ZV):
                idxb[pl.ds(v * 16, 16)] = iota16 + (rbase + v * 16)
            pltpu.sync_copy(tas[p], acc.at[idxb], add=True)
            znxt = jnp.minimum(z + 2, ZBLKS - 1)
            issue_inputs(znxt, p)

        issue_inputs(0, 0)
        issue_inputs(1, 1)

        def mgrp(g, carry):
            handle(2 * g, 0)
            handle(2 * g + 1, 1)
            return carry
        lax.fori_loop(0, ZBLKS // 2, mgrp, 0)
        for p in range(2):
            slh = pl.ds(cN + stripe0 + (ZBLKS - 1) * ZROWS, ZROWS)
            pltpu.make_async_copy(t0.at[slh], tas[p], msems[p]).wait()
            pltpu.make_async_copy(e1.at[slh], tbs[p], msems[p]).wait()

    for k, (tsrc, tdst) in enumerate(((t0, e1), (e1, e2), (e2, None))):
        last = k == K - 1
        if last:
            mean_add_phase()
        for b in range(3):
            pltpu.async_copy(packed.at[base + b], ibufs[b], isems[b])
            pltpu.async_copy(wtab.at[base + b], wbufs[b], isems[b])
        pltpu.make_async_copy(packed.at[base], ibufs[0], isems[0]).wait()
        pltpu.make_async_copy(wtab.at[base], wbufs[0], isems[0]).wait()
        prep(ibufs[0], wbufs[0], last)
        pltpu.async_copy(tsrc.at[ibufs[0].at[0]], rbufs[0], gsems[0])

        def group(g, carry):
            for b in range(4):
                i = 4 * g + b
                bn1, bn3 = (b + 1) % 4, (b + 3) % 4
                rn1 = (b + 1) % 2
                pltpu.make_async_copy(
                    packed.at[base], ibufs[bn1], isems[bn1]).wait()
                pltpu.make_async_copy(
                    wtab.at[base], wbufs[bn1], isems[bn1]).wait()
                prep(ibufs[bn1], wbufs[bn1], last)
                if b == 0:
                    @pl.when(g > 0)
                    def _():
                        pltpu.make_async_copy(
                            rbufs[rn1], acc.at[ibufs[bn3].at[1]],
                            ssems[rn1]).wait()
                else:
                    pltpu.make_async_copy(
                        rbufs[rn1], acc.at[ibufs[bn3].at[1]],
                        ssems[rn1]).wait()
                cid = base + jnp.minimum(i + 3, NCH_TILE - 1)
                pltpu.async_copy(packed.at[cid], ibufs[bn3], isems[bn3])
                pltpu.async_copy(wtab.at[cid], wbufs[bn3], isems[bn3])
                pltpu.async_copy(
                    tsrc.at[ibufs[bn1].at[0]], rbufs[rn1], gsems[rn1])
                pltpu.make_async_copy(
                    tsrc.at[ibufs[b].at[0]], rbufs[b % 2],
                    gsems[b % 2]).wait()
                scale(rbufs[b % 2], wbufs[b])
                pltpu.async_copy(
                    rbufs[b % 2], acc.at[ibufs[b].at[1]], ssems[b % 2],
                    add=True)
            return carry
        lax.fori_loop(0, GROUPS, group, 0)

        pltpu.make_async_copy(rbufs[1], acc.at[ibufs[3].at[1]], ssems[1]).wait()
        pltpu.make_async_copy(tsrc.at[ibufs[0].at[0]], rbufs[0],
                              gsems[0]).wait()
        pltpu.make_async_copy(packed.at[base], ibufs[1], isems[1]).wait()
        pltpu.make_async_copy(wtab.at[base], wbufs[1], isems[1]).wait()
        pltpu.make_async_copy(packed.at[base], ibufs[2], isems[2]).wait()
        pltpu.make_async_copy(wtab.at[base], wbufs[2], isems[2]).wait()
        plsc.subcore_barrier()

        out_ref = favg if last else tdst
        for z in range(ZBLKS):
            sl = pl.ds(stripe0 + z * ZROWS, ZROWS)
            pltpu.sync_copy(acc.at[sl],
                            out_ref.at[pl.ds(cN + stripe0 + z * ZROWS, ZROWS)])
            if not last:
                pltpu.sync_copy(zbuf, acc.at[sl])
        plsc.subcore_barrier()


_sc_prop = functools.partial(
    pl.kernel,
    out_type=[jax.ShapeDtypeStruct((2 * NP, HALF), jnp.float32)] * 3,
    mesh=plsc.VectorSubcoreMesh(core_axis_name="c", subcore_axis_name="s"),
    compiler_params=pltpu.CompilerParams(
        needs_layout_passes=False, use_tc_tiling_on_sc=False),
    scratch_types=[pltpu.VMEM((2, CH), jnp.int32)] * 4
                  + [pltpu.VMEM((CH,), jnp.float32)] * 4
                  + [pltpu.VMEM((CH, HALF), jnp.float32)] * 2
                  + [pltpu.VMEM((ZROWS, HALF), jnp.float32)]
                  + [pltpu.VMEM_SHARED((NP, HALF), jnp.float32)]
                  + [pltpu.VMEM((ZROWS, HALF), jnp.float32)] * 4
                  + [pltpu.VMEM((ZROWS,), jnp.int32)]
                  + [pltpu.SemaphoreType.DMA] * 10,
)(_sc_body)


def kernel(edge_index, edge_weight, users_w, items_w):
    col = edge_index[1].astype(jnp.int32)
    row = edge_index[0].astype(jnp.int32)
    w = edge_weight.astype(jnp.float32)
    pad = EP - E
    col = jnp.concatenate([col, jnp.zeros((pad,), jnp.int32)])
    row = jnp.concatenate([row, jnp.zeros((pad,), jnp.int32)])
    w = jnp.concatenate([w, jnp.zeros((pad,), jnp.float32)])
    packed = jnp.stack(
        [col.reshape(NCHUNKS, CH), row.reshape(NCHUNKS, CH)],
        axis=1)
    wtab = w.reshape(NCHUNKS, CH)
    emb0 = jnp.concatenate(
        [users_w, items_w, jnp.zeros((NP - N, EMB), jnp.float32)], axis=0)
    t0 = jnp.concatenate([emb0[:, :HALF], emb0[:, HALF:]], axis=0)
    favg, _e1, _e2 = _sc_prop(packed, wtab, t0)
    final = jnp.concatenate([favg[:N], favg[NP:NP + N]], axis=1)
    return final[:NUM_USERS], users_w, final[NUM_USERS:], items_w
